# Initial kernel scaffold; baseline (speedup 1.0000x reference)
import functools

import jax
import jax.numpy as jnp
from jax import lax
from jax.experimental import pallas as pl
from jax.experimental.pallas import tpu as pltpu
from jax.experimental.pallas import tpu_sc as plsc

_NC = 2
_NS = 16
_NW = _NC * _NS

_CH = 80


def _edge_agg_kernel(n_nodes, n_pad, n_edges, d):
  e_tile = n_edges // _NW
  n_chunks = e_tile // _CH
  slab = n_pad // _NS
  mesh = plsc.VectorSubcoreMesh(core_axis_name="c", subcore_axis_name="s")

  @functools.partial(
      pl.kernel,
      out_type=jax.ShapeDtypeStruct((_NC, n_pad, d), jnp.float32),
      mesh=mesh,
      scratch_types=[
          pltpu.VMEM((_CH,), jnp.int32),
          pltpu.VMEM((_CH,), jnp.int32),
          pltpu.VMEM((_CH, d), jnp.float32),
          pltpu.VMEM_SHARED((n_pad, d), jnp.float32),
          pltpu.SemaphoreType.DMA,
      ],
  )
  def k(hs_hbm, src_hbm, dst_hbm, zeros_hbm, out_hbm, idx_s, idx_d, rows,
        acc, sem):
    c = lax.axis_index("c")
    s = lax.axis_index("s")
    wid = c * _NS + s
    pltpu.sync_copy(zeros_hbm, acc.at[pl.ds(s * slab, slab)])
    plsc.subcore_barrier()

    base = wid * e_tile

    @pl.loop(0, n_chunks)
    def _chunk(j):
      off = base + j * _CH
      pltpu.sync_copy(src_hbm.at[pl.ds(off, _CH)], idx_s)
      pltpu.sync_copy(dst_hbm.at[pl.ds(off, _CH)], idx_d)
      pltpu.async_copy(hs_hbm.at[idx_s], rows, sem).wait()
      pltpu.sync_copy(rows, acc.at[idx_d], add=True)

    plsc.subcore_barrier()
    pltpu.sync_copy(acc.at[pl.ds(s * slab, slab)],
                    out_hbm.at[c, pl.ds(s * slab, slab)])

  return k


def _degree_kernel(n_pad, n_edges):
  e_tile = n_edges // _NW
  n_chunks = e_tile // _CH
  slab = n_pad // _NS
  mesh = plsc.VectorSubcoreMesh(core_axis_name="c", subcore_axis_name="s")

  @functools.partial(
      pl.kernel,
      out_type=jax.ShapeDtypeStruct((_NC, n_pad, 8), jnp.float32),
      mesh=mesh,
      scratch_types=[
          pltpu.VMEM((_CH,), jnp.int32),
          pltpu.VMEM((_CH, 8), jnp.float32),
          pltpu.VMEM_SHARED((n_pad, 8), jnp.float32),
      ],
  )
  def k(dst_hbm, ones_hbm, zeros_hbm, out_hbm, idx_d, ones_v, acc):
    c = lax.axis_index("c")
    s = lax.axis_index("s")
    wid = c * _NS + s
    pltpu.sync_copy(ones_hbm, ones_v)
    pltpu.sync_copy(zeros_hbm, acc.at[pl.ds(s * slab, slab)])
    plsc.subcore_barrier()

    base = wid * e_tile

    @pl.loop(0, n_chunks)
    def _chunk(j):
      off = base + j * _CH
      pltpu.sync_copy(dst_hbm.at[pl.ds(off, _CH)], idx_d)
      pltpu.sync_copy(ones_v, acc.at[idx_d], add=True)

    plsc.subcore_barrier()
    pltpu.sync_copy(acc.at[pl.ds(s * slab, slab)],
                    out_hbm.at[c, pl.ds(s * slab, slab)])

  return k


def _tc_first(x_ref, w_ref, degp_ref, out_ref):
  n = x_ref.shape[0]
  deg = degp_ref[0, :n, 0:1] + degp_ref[1, :n, 0:1] + 1.0
  g = lax.rsqrt(deg)
  h = jnp.dot(x_ref[...], w_ref[...], preferred_element_type=jnp.float32)
  out_ref[...] = h * g


def _tc_mid(aggp_ref, hs_ref, degp_ref, b_ref, w_ref, out_ref):
  n = hs_ref.shape[0]
  deg = degp_ref[0, :n, 0:1] + degp_ref[1, :n, 0:1] + 1.0
  g = lax.rsqrt(deg)
  agg = aggp_ref[0, :n, :] + aggp_ref[1, :n, :] + hs_ref[...]
  h = jnp.maximum(agg * g + b_ref[...], 0.0)
  out_ref[...] = jnp.dot(h, w_ref[...],
                         preferred_element_type=jnp.float32) * g


def _tc_last(aggp_ref, hs_ref, degp_ref, b_ref, batch_ref, out_ref,
             *, num_graphs):
  n = hs_ref.shape[0]
  deg = degp_ref[0, :n, 0:1] + degp_ref[1, :n, 0:1] + 1.0
  g = lax.rsqrt(deg)
  nodes = (aggp_ref[0, :n, :] + aggp_ref[1, :n, :] + hs_ref[...]) * g \
      + b_ref[...]
  gid = lax.broadcasted_iota(jnp.int32, (num_graphs, n), 0)
  mask = (gid == batch_ref[...]).astype(jnp.float32)
  sums = jnp.dot(mask, nodes, preferred_element_type=jnp.float32)
  cnt = jnp.sum(mask, axis=1, keepdims=True)
  out_ref[...] = sums / jnp.maximum(cnt, 1.0)


def kernel(x, edge_index, batch, W1, b1, W2, b2):
  n, d_in = x.shape
  d_hid = W1.shape[1]
  d_out = W2.shape[1]
  e = edge_index.shape[1]
  num_graphs = 64

  n_pad = ((n + (8 * _NS) - 1) // (8 * _NS)) * (8 * _NS)
  slab = n_pad // _NS

  src = edge_index[0]
  dst = edge_index[1]
  zeros_d = jnp.zeros((slab, d_hid), jnp.float32)
  zeros_8 = jnp.zeros((slab, 8), jnp.float32)
  ones_8 = jnp.ones((_CH, 8), jnp.float32)
  batch2d = batch.reshape(1, n)
  b1r = b1.reshape(1, d_hid)
  b2r = b2.reshape(1, d_out)

  deg_p = _degree_kernel(n_pad, e)(dst, ones_8, zeros_8)

  agg = _edge_agg_kernel(n, n_pad, e, d_hid)

  hs1 = pl.pallas_call(
      _tc_first,
      out_shape=jax.ShapeDtypeStruct((n, d_hid), jnp.float32),
  )(x, W1, deg_p)

  agg1 = agg(hs1, src, dst, zeros_d)

  hs2 = pl.pallas_call(
      _tc_mid,
      out_shape=jax.ShapeDtypeStruct((n, d_out), jnp.float32),
  )(agg1, hs1, deg_p, b1r, W2)

  agg2 = agg(hs2, src, dst, zeros_d)

  out = pl.pallas_call(
      functools.partial(_tc_last, num_graphs=num_graphs),
      out_shape=jax.ShapeDtypeStruct((num_graphs, d_out), jnp.float32),
  )(agg2, hs2, deg_p, b2r, batch2d)

  return out


# speedup vs baseline: 13.2584x; 13.2584x over previous
import functools

import jax
import jax.numpy as jnp
from jax import lax
from jax.experimental import pallas as pl
from jax.experimental.pallas import tpu as pltpu
from jax.experimental.pallas import tpu_sc as plsc

_NC = 2
_NS = 16
_NW = _NC * _NS

_CH = 80


def _edge_agg_kernel(n_nodes, n_pad, n_edges, d):
  e_tile = n_edges // _NW
  n_chunks = e_tile // _CH
  slab = n_pad // _NS
  mesh = plsc.VectorSubcoreMesh(core_axis_name="c", subcore_axis_name="s")

  @functools.partial(
      pl.kernel,
      out_type=jax.ShapeDtypeStruct((_NC, n_pad, d), jnp.float32),
      mesh=mesh,
      scratch_types=[
          pltpu.VMEM((_CH,), jnp.int32),
          pltpu.VMEM((_CH,), jnp.int32),
          pltpu.VMEM((_CH, d), jnp.float32),
          pltpu.VMEM_SHARED((n_pad, d), jnp.float32),
          pltpu.SemaphoreType.DMA,
      ],
  )
  def k(hs_hbm, src_hbm, dst_hbm, zeros_hbm, out_hbm, idx_s, idx_d, rows,
        acc, sem):
    c = lax.axis_index("c")
    s = lax.axis_index("s")
    wid = c * _NS + s
    pltpu.sync_copy(zeros_hbm, acc.at[pl.ds(s * slab, slab)])
    plsc.subcore_barrier()

    base = wid * e_tile

    @pl.loop(0, n_chunks)
    def _chunk(j):
      off = base + j * _CH
      pltpu.sync_copy(src_hbm.at[pl.ds(off, _CH)], idx_s)
      pltpu.sync_copy(dst_hbm.at[pl.ds(off, _CH)], idx_d)
      pltpu.async_copy(hs_hbm.at[idx_s], rows, sem).wait()
      pltpu.sync_copy(rows, acc.at[idx_d], add=True)

    plsc.subcore_barrier()
    pltpu.sync_copy(acc.at[pl.ds(s * slab, slab)],
                    out_hbm.at[c, pl.ds(s * slab, slab)])

  return k


def _degree_kernel(n_pad, n_edges):
  e_tile = n_edges // _NW
  n_chunks = e_tile // _CH
  slab = n_pad // _NS
  mesh = plsc.VectorSubcoreMesh(core_axis_name="c", subcore_axis_name="s")

  @functools.partial(
      pl.kernel,
      out_type=jax.ShapeDtypeStruct((_NC, n_pad), jnp.float32),
      mesh=mesh,
      scratch_types=[
          pltpu.VMEM((_CH,), jnp.int32),
          pltpu.VMEM((_CH,), jnp.float32),
          pltpu.VMEM_SHARED((n_pad,), jnp.float32),
      ],
  )
  def k(dst_hbm, ones_hbm, zeros_hbm, out_hbm, idx_d, ones_v, acc):
    c = lax.axis_index("c")
    s = lax.axis_index("s")
    wid = c * _NS + s
    pltpu.sync_copy(ones_hbm, ones_v)
    pltpu.sync_copy(zeros_hbm.at[pl.ds(0, slab)],
                    acc.at[pl.ds(s * slab, slab)])
    plsc.subcore_barrier()

    @pl.loop(0, n_chunks)
    def _chunk(j):
      off = wid * e_tile + j * _CH
      pltpu.sync_copy(dst_hbm.at[pl.ds(off, _CH)], idx_d)
      pltpu.sync_copy(ones_v, acc.at[idx_d], add=True)

    plsc.subcore_barrier()
    pltpu.sync_copy(acc.at[pl.ds(s * slab, slab)],
                    out_hbm.at[c, pl.ds(s * slab, slab)])

  return k


def _tc_first(x_ref, w_ref, degp_ref, out_ref):
  n = x_ref.shape[0]
  deg = degp_ref[0, :n, 0:1] + degp_ref[1, :n, 0:1] + 1.0
  g = lax.rsqrt(deg)
  h = jnp.dot(x_ref[...], w_ref[...], preferred_element_type=jnp.float32)
  out_ref[...] = h * g


def _tc_mid(aggp_ref, hs_ref, degp_ref, b_ref, w_ref, out_ref):
  n = hs_ref.shape[0]
  deg = degp_ref[0, :n, 0:1] + degp_ref[1, :n, 0:1] + 1.0
  g = lax.rsqrt(deg)
  agg = aggp_ref[0, :n, :] + aggp_ref[1, :n, :] + hs_ref[...]
  h = jnp.maximum(agg * g + b_ref[...], 0.0)
  out_ref[...] = jnp.dot(h, w_ref[...],
                         preferred_element_type=jnp.float32) * g


def _tc_last(aggp_ref, hs_ref, degp_ref, b_ref, batch_ref, out_ref,
             *, num_graphs):
  n = hs_ref.shape[0]
  deg = degp_ref[0, :n, 0:1] + degp_ref[1, :n, 0:1] + 1.0
  g = lax.rsqrt(deg)
  nodes = (aggp_ref[0, :n, :] + aggp_ref[1, :n, :] + hs_ref[...]) * g \
      + b_ref[...]
  gid = lax.broadcasted_iota(jnp.int32, (num_graphs, n), 0)
  mask = (gid == batch_ref[...]).astype(jnp.float32)
  sums = jnp.dot(mask, nodes, preferred_element_type=jnp.float32)
  cnt = jnp.sum(mask, axis=1, keepdims=True)
  out_ref[...] = sums / jnp.maximum(cnt, 1.0)


def kernel(x, edge_index, batch, W1, b1, W2, b2):
  n, d_in = x.shape
  d_hid = W1.shape[1]
  d_out = W2.shape[1]
  e = edge_index.shape[1]
  num_graphs = 64

  n_pad = ((n + 1023) // 1024) * 1024
  slab = n_pad // _NS

  src = edge_index[0]
  dst = edge_index[1]
  zeros_d = jnp.zeros((slab, d_hid), jnp.float32)
  zeros_1d = jnp.zeros((n_pad,), jnp.float32)
  ones_ch = jnp.ones((_CH,), jnp.float32)
  batch2d = batch.reshape(1, n)
  b1r = b1.reshape(1, d_hid)
  b2r = b2.reshape(1, d_out)

  deg_raw = _degree_kernel(n_pad, e)(dst, ones_ch, zeros_1d)
  deg_p = deg_raw.reshape(_NC, n_pad, 1)

  agg = _edge_agg_kernel(n, n_pad, e, d_hid)

  hs1 = pl.pallas_call(
      _tc_first,
      out_shape=jax.ShapeDtypeStruct((n, d_hid), jnp.float32),
  )(x, W1, deg_p)

  agg1 = agg(hs1, src, dst, zeros_d)

  hs2 = pl.pallas_call(
      _tc_mid,
      out_shape=jax.ShapeDtypeStruct((n, d_out), jnp.float32),
  )(agg1, hs1, deg_p, b1r, W2)

  agg2 = agg(hs2, src, dst, zeros_d)

  out = pl.pallas_call(
      functools.partial(_tc_last, num_graphs=num_graphs),
      out_shape=jax.ShapeDtypeStruct((num_graphs, d_out), jnp.float32),
  )(agg2, hs2, deg_p, b2r, batch2d)

  return out


# speedup vs baseline: 18.4778x; 1.3937x over previous
import functools

import jax
import jax.numpy as jnp
from jax import lax
from jax.experimental import pallas as pl
from jax.experimental.pallas import tpu as pltpu
from jax.experimental.pallas import tpu_sc as plsc

_NC = 2
_NS = 16
_NW = _NC * _NS

_CH = 96
_NB = 2


def _edge_agg_kernel(n_pad, e_pad, d):
  e_tile = e_pad // _NW
  n_ch = e_tile // _CH
  slab = n_pad // _NS
  mesh = plsc.VectorSubcoreMesh(core_axis_name="c", subcore_axis_name="s")

  nbi = 4

  @functools.partial(
      pl.kernel,
      out_type=jax.ShapeDtypeStruct((_NC, n_pad, d), jnp.float32),
      mesh=mesh,
      scratch_types=[
          pltpu.VMEM((nbi, _CH), jnp.int32),
          pltpu.VMEM((n_ch, _CH), jnp.int32),
          pltpu.VMEM((_NB, _CH, d), jnp.float32),
          pltpu.VMEM_SHARED((n_pad, d), jnp.float32),
          pltpu.SemaphoreType.DMA((_NB,)),
          pltpu.SemaphoreType.DMA((nbi,)),
      ],
  )
  def k(hs_hbm, src_hbm, dst_hbm, zeros_hbm, out_hbm, idx_s, idx_d, rows,
        acc, sem_g, sem_i):
    c = lax.axis_index("c")
    s = lax.axis_index("s")
    wid = c * _NS + s
    pltpu.sync_copy(dst_hbm.at[wid], idx_d)
    pltpu.sync_copy(zeros_hbm, acc.at[pl.ds(s * slab, slab)])
    for bi in range(nbi):
      pltpu.async_copy(src_hbm.at[wid, bi], idx_s.at[bi], sem_i.at[bi])
    plsc.subcore_barrier()

    for b in range(_NB):
      pltpu.make_async_copy(src_hbm.at[wid, b], idx_s.at[b],
                            sem_i.at[b]).wait()
      pltpu.async_copy(hs_hbm.at[idx_s.at[b]], rows.at[b], sem_g.at[b])

    @pl.loop(0, n_ch)
    def _chunk(j):
      b = lax.rem(j, _NB)
      bi = lax.rem(j, nbi)
      pltpu.make_async_copy(hs_hbm.at[idx_s.at[bi]], rows.at[b],
                            sem_g.at[b]).wait()
      pltpu.sync_copy(rows.at[b], acc.at[idx_d.at[j]], add=True)
      nxt4 = j + nbi

      @pl.when(nxt4 < n_ch)
      def _pref_idx():
        pltpu.async_copy(src_hbm.at[wid, nxt4], idx_s.at[bi], sem_i.at[bi])

      nxt2 = j + _NB

      @pl.when(nxt2 < n_ch)
      def _pref_rows():
        bi2 = lax.rem(nxt2, nbi)
        pltpu.make_async_copy(src_hbm.at[wid, nxt2], idx_s.at[bi2],
                              sem_i.at[bi2]).wait()
        pltpu.async_copy(hs_hbm.at[idx_s.at[bi2]], rows.at[b], sem_g.at[b])

    plsc.subcore_barrier()
    pltpu.sync_copy(acc.at[pl.ds(s * slab, slab)],
                    out_hbm.at[c, pl.ds(s * slab, slab)])

  return k


def _degree_kernel(n_pad, n_edges):
  e_tile = n_edges // _NW
  n_ch = e_tile // _CH
  slab = n_pad // _NS
  nbd = 8
  mesh = plsc.VectorSubcoreMesh(core_axis_name="c", subcore_axis_name="s")

  @functools.partial(
      pl.kernel,
      out_type=jax.ShapeDtypeStruct((_NC, n_pad), jnp.float32),
      mesh=mesh,
      scratch_types=[
          pltpu.VMEM((n_ch, _CH), jnp.int32),
          pltpu.VMEM((_CH,), jnp.float32),
          pltpu.VMEM_SHARED((n_pad,), jnp.float32),
          pltpu.SemaphoreType.DMA((nbd,)),
      ],
  )
  def k(dst_hbm, ones_hbm, zeros_hbm, out_hbm, idx_d, ones_v, acc, sem_s):
    c = lax.axis_index("c")
    s = lax.axis_index("s")
    wid = c * _NS + s
    pltpu.sync_copy(dst_hbm.at[wid], idx_d)
    pltpu.sync_copy(ones_hbm, ones_v)
    pltpu.sync_copy(zeros_hbm.at[pl.ds(0, slab)],
                    acc.at[pl.ds(s * slab, slab)])
    plsc.subcore_barrier()

    for b in range(nbd):
      pltpu.async_copy(ones_v, acc.at[idx_d.at[b]], sem_s.at[b], add=True)

    @pl.loop(0, n_ch)
    def _chunk(j):
      b = lax.rem(j, nbd)
      pltpu.make_async_copy(ones_v, acc.at[idx_d.at[j]], sem_s.at[b]).wait()
      nxt = j + nbd

      @pl.when(nxt < n_ch)
      def _nx():
        pltpu.async_copy(ones_v, acc.at[idx_d.at[nxt]], sem_s.at[b],
                         add=True)

    plsc.subcore_barrier()
    pltpu.sync_copy(acc.at[pl.ds(s * slab, slab)],
                    out_hbm.at[c, pl.ds(s * slab, slab)])

  return k


def _tc_first(x_ref, w_ref, degp_ref, out_ref):
  n = x_ref.shape[0]
  deg = degp_ref[0, :n, 0:1] + degp_ref[1, :n, 0:1] + 1.0
  g = lax.rsqrt(deg)
  h = jnp.dot(x_ref[...], w_ref[...], preferred_element_type=jnp.float32)
  out_ref[...] = h * g


def _tc_mid(aggp_ref, hs_ref, degp_ref, b_ref, w_ref, out_ref):
  n = hs_ref.shape[0]
  deg = degp_ref[0, :n, 0:1] + degp_ref[1, :n, 0:1] + 1.0
  g = lax.rsqrt(deg)
  agg = aggp_ref[0, :n, :] + aggp_ref[1, :n, :] + hs_ref[...]
  h = jnp.maximum(agg * g + b_ref[...], 0.0)
  out_ref[...] = jnp.dot(h, w_ref[...],
                         preferred_element_type=jnp.float32) * g


def _tc_last(aggp_ref, hs_ref, degp_ref, b_ref, batch_ref, out_ref,
             *, num_graphs):
  n = hs_ref.shape[0]
  deg = degp_ref[0, :n, 0:1] + degp_ref[1, :n, 0:1] + 1.0
  g = lax.rsqrt(deg)
  nodes = (aggp_ref[0, :n, :] + aggp_ref[1, :n, :] + hs_ref[...]) * g \
      + b_ref[...]
  gid = lax.broadcasted_iota(jnp.int32, (num_graphs, n), 0)
  mask = (gid == batch_ref[...]).astype(jnp.float32)
  sums = jnp.dot(mask, nodes, preferred_element_type=jnp.float32)
  cnt = jnp.sum(mask, axis=1, keepdims=True)
  out_ref[...] = sums / jnp.maximum(cnt, 1.0)


def kernel(x, edge_index, batch, W1, b1, W2, b2):
  n, d_in = x.shape
  d_hid = W1.shape[1]
  d_out = W2.shape[1]
  e = edge_index.shape[1]
  num_graphs = 64

  n_pad = ((n + 1023) // 1024) * 1024
  slab = n_pad // _NS

  grain = _NW * _CH
  e_pad = ((e + grain - 1) // grain) * grain
  src_p = jnp.concatenate(
      [edge_index[0], jnp.zeros((e_pad - e,), jnp.int32)]
  ).reshape(_NW, -1, _CH)
  dst_p = jnp.concatenate(
      [edge_index[1], jnp.full((e_pad - e,), n_pad - 1, jnp.int32)]
  ).reshape(_NW, -1, _CH)

  zeros_d = jnp.zeros((slab, d_hid), jnp.float32)
  zeros_1d = jnp.zeros((n_pad,), jnp.float32)
  ones_ch = jnp.ones((_CH,), jnp.float32)
  batch2d = batch.reshape(1, n)
  b1r = b1.reshape(1, d_hid)
  b2r = b2.reshape(1, d_out)

  deg_raw = _degree_kernel(n_pad, e_pad)(dst_p, ones_ch, zeros_1d)
  deg_p = deg_raw.reshape(_NC, n_pad, 1)

  agg = _edge_agg_kernel(n_pad, e_pad, d_hid)

  hs1 = pl.pallas_call(
      _tc_first,
      out_shape=jax.ShapeDtypeStruct((n, d_hid), jnp.float32),
  )(x, W1, deg_p)

  agg1 = agg(hs1, src_p, dst_p, zeros_d)

  hs2 = pl.pallas_call(
      _tc_mid,
      out_shape=jax.ShapeDtypeStruct((n, d_out), jnp.float32),
  )(agg1, hs1, deg_p, b1r, W2)

  agg2 = agg(hs2, src_p, dst_p, zeros_d)

  out = pl.pallas_call(
      functools.partial(_tc_last, num_graphs=num_graphs),
      out_shape=jax.ShapeDtypeStruct((num_graphs, d_out), jnp.float32),
  )(agg2, hs2, deg_p, b2r, batch2d)

  return out


# speedup vs baseline: 19.1847x; 1.0383x over previous
import functools

import jax
import jax.numpy as jnp
from jax import lax
from jax.experimental import pallas as pl
from jax.experimental.pallas import tpu as pltpu
from jax.experimental.pallas import tpu_sc as plsc

_NC = 2
_NS = 16
_NW = _NC * _NS

_CH = 96
_NB = 3


def _edge_agg_kernel(n_pad, e_pad, d):
  e_tile = e_pad // _NW
  n_ch = e_tile // _CH
  slab = n_pad // _NS
  mesh = plsc.VectorSubcoreMesh(core_axis_name="c", subcore_axis_name="s")

  nbi = 6
  nbd = 8

  @functools.partial(
      pl.kernel,
      out_type=jax.ShapeDtypeStruct((_NC, n_pad, d), jnp.float32),
      mesh=mesh,
      scratch_types=[
          pltpu.VMEM((nbi, _CH), jnp.int32),
          pltpu.VMEM((nbd, _CH), jnp.int32),
          pltpu.VMEM((_NB, _CH, d), jnp.float32),
          pltpu.VMEM_SHARED((n_pad, d), jnp.float32),
          pltpu.SemaphoreType.DMA((_NB,)),
          pltpu.SemaphoreType.DMA((_NB,)),
          pltpu.SemaphoreType.DMA((nbi,)),
          pltpu.SemaphoreType.DMA((nbd,)),
      ],
  )
  def k(hs_hbm, src_hbm, dst_hbm, zeros_hbm, out_hbm, idx_s, idx_d, rows,
        acc, sem_g, sem_s, sem_i, sem_j):
    c = lax.axis_index("c")
    s = lax.axis_index("s")
    wid = c * _NS + s
    pltpu.sync_copy(zeros_hbm, acc.at[pl.ds(s * slab, slab)])
    for bi in range(nbi):
      pltpu.async_copy(src_hbm.at[wid, bi], idx_s.at[bi], sem_i.at[bi])
    for bj in range(nbd):
      pltpu.async_copy(dst_hbm.at[wid, bj], idx_d.at[bj], sem_j.at[bj])
    plsc.subcore_barrier()

    for b in range(_NB - 1):
      pltpu.make_async_copy(src_hbm.at[wid, b], idx_s.at[b],
                            sem_i.at[b]).wait()
      pltpu.async_copy(hs_hbm.at[idx_s.at[b]], rows.at[b], sem_g.at[b])

    @pl.loop(0, n_ch)
    def _chunk(j):
      b = lax.rem(j, _NB)
      bi = lax.rem(j, nbi)
      bj = lax.rem(j, nbd)
      pltpu.make_async_copy(hs_hbm.at[idx_s.at[bi]], rows.at[b],
                            sem_g.at[b]).wait()
      prv = j - 1

      @pl.when(prv >= 0)
      def _wprev():
        bp = lax.rem(prv, _NB)
        bjp = lax.rem(prv, nbd)
        pltpu.make_async_copy(rows.at[bp], acc.at[idx_d.at[bjp]],
                              sem_s.at[bp]).wait()
        nxj = prv + nbd

        @pl.when(nxj < n_ch)
        def _pref_di():
          pltpu.async_copy(dst_hbm.at[wid, nxj], idx_d.at[bjp],
                           sem_j.at[bjp])

      pltpu.make_async_copy(dst_hbm.at[wid, j], idx_d.at[bj],
                            sem_j.at[bj]).wait()
      pltpu.async_copy(rows.at[b], acc.at[idx_d.at[bj]], sem_s.at[b],
                       add=True)
      nxi = j + nbi

      @pl.when(nxi < n_ch)
      def _pref_si():
        pltpu.async_copy(src_hbm.at[wid, nxi], idx_s.at[bi], sem_i.at[bi])

      nxg = j + _NB - 1

      @pl.when(nxg < n_ch)
      def _pref_rows():
        bg = lax.rem(nxg, _NB)
        big = lax.rem(nxg, nbi)
        pltpu.make_async_copy(src_hbm.at[wid, nxg], idx_s.at[big],
                              sem_i.at[big]).wait()
        pltpu.async_copy(hs_hbm.at[idx_s.at[big]], rows.at[bg],
                         sem_g.at[bg])

    lst = n_ch - 1
    pltpu.make_async_copy(rows.at[lax.rem(lst, _NB)],
                          acc.at[idx_d.at[lax.rem(lst, nbd)]],
                          sem_s.at[lax.rem(lst, _NB)]).wait()
    plsc.subcore_barrier()
    pltpu.sync_copy(acc.at[pl.ds(s * slab, slab)],
                    out_hbm.at[c, pl.ds(s * slab, slab)])

  return k


def _degree_kernel(n_pad, n_edges):
  e_tile = n_edges // _NW
  n_ch = e_tile // _CH
  slab = n_pad // _NS
  nbd = 8
  mesh = plsc.VectorSubcoreMesh(core_axis_name="c", subcore_axis_name="s")

  @functools.partial(
      pl.kernel,
      out_type=jax.ShapeDtypeStruct((_NC, n_pad), jnp.float32),
      mesh=mesh,
      scratch_types=[
          pltpu.VMEM((n_ch, _CH), jnp.int32),
          pltpu.VMEM((_CH,), jnp.float32),
          pltpu.VMEM_SHARED((n_pad,), jnp.float32),
          pltpu.SemaphoreType.DMA((nbd,)),
      ],
  )
  def k(dst_hbm, ones_hbm, zeros_hbm, out_hbm, idx_d, ones_v, acc, sem_s):
    c = lax.axis_index("c")
    s = lax.axis_index("s")
    wid = c * _NS + s
    pltpu.sync_copy(dst_hbm.at[wid], idx_d)
    pltpu.sync_copy(ones_hbm, ones_v)
    pltpu.sync_copy(zeros_hbm.at[pl.ds(0, slab)],
                    acc.at[pl.ds(s * slab, slab)])
    plsc.subcore_barrier()

    for b in range(nbd):
      pltpu.async_copy(ones_v, acc.at[idx_d.at[b]], sem_s.at[b], add=True)

    @pl.loop(0, n_ch)
    def _chunk(j):
      b = lax.rem(j, nbd)
      pltpu.make_async_copy(ones_v, acc.at[idx_d.at[j]], sem_s.at[b]).wait()
      nxt = j + nbd

      @pl.when(nxt < n_ch)
      def _nx():
        pltpu.async_copy(ones_v, acc.at[idx_d.at[nxt]], sem_s.at[b],
                         add=True)

    plsc.subcore_barrier()
    pltpu.sync_copy(acc.at[pl.ds(s * slab, slab)],
                    out_hbm.at[c, pl.ds(s * slab, slab)])

  return k


def _tc_first(x_ref, w_ref, degp_ref, out_ref):
  n = x_ref.shape[0]
  deg = degp_ref[0, :n, 0:1] + degp_ref[1, :n, 0:1] + 1.0
  g = lax.rsqrt(deg)
  h = jnp.dot(x_ref[...], w_ref[...], preferred_element_type=jnp.float32)
  out_ref[...] = h * g


def _tc_mid(aggp_ref, hs_ref, degp_ref, b_ref, w_ref, out_ref):
  n = hs_ref.shape[0]
  deg = degp_ref[0, :n, 0:1] + degp_ref[1, :n, 0:1] + 1.0
  g = lax.rsqrt(deg)
  agg = aggp_ref[0, :n, :] + aggp_ref[1, :n, :] + hs_ref[...]
  h = jnp.maximum(agg * g + b_ref[...], 0.0)
  out_ref[...] = jnp.dot(h, w_ref[...],
                         preferred_element_type=jnp.float32) * g


def _tc_last(aggp_ref, hs_ref, degp_ref, b_ref, batch_ref, out_ref,
             *, num_graphs):
  n = hs_ref.shape[0]
  deg = degp_ref[0, :n, 0:1] + degp_ref[1, :n, 0:1] + 1.0
  g = lax.rsqrt(deg)
  nodes = (aggp_ref[0, :n, :] + aggp_ref[1, :n, :] + hs_ref[...]) * g \
      + b_ref[...]
  gid = lax.broadcasted_iota(jnp.int32, (num_graphs, n), 0)
  mask = (gid == batch_ref[...]).astype(jnp.float32)
  sums = jnp.dot(mask, nodes, preferred_element_type=jnp.float32)
  cnt = jnp.sum(mask, axis=1, keepdims=True)
  out_ref[...] = sums / jnp.maximum(cnt, 1.0)


def kernel(x, edge_index, batch, W1, b1, W2, b2):
  n, d_in = x.shape
  d_hid = W1.shape[1]
  d_out = W2.shape[1]
  e = edge_index.shape[1]
  num_graphs = 64

  n_pad = ((n + 1023) // 1024) * 1024
  slab = n_pad // _NS

  grain = _NW * _CH
  e_pad = ((e + grain - 1) // grain) * grain
  src_p = jnp.concatenate(
      [edge_index[0], jnp.zeros((e_pad - e,), jnp.int32)]
  ).reshape(_NW, -1, _CH)
  dst_p = jnp.concatenate(
      [edge_index[1], jnp.full((e_pad - e,), n_pad - 1, jnp.int32)]
  ).reshape(_NW, -1, _CH)

  zeros_d = jnp.zeros((slab, d_hid), jnp.float32)
  zeros_1d = jnp.zeros((n_pad,), jnp.float32)
  ones_ch = jnp.ones((_CH,), jnp.float32)
  batch2d = batch.reshape(1, n)
  b1r = b1.reshape(1, d_hid)
  b2r = b2.reshape(1, d_out)

  deg_raw = _degree_kernel(n_pad, e_pad)(dst_p, ones_ch, zeros_1d)
  deg_p = deg_raw.reshape(_NC, n_pad, 1)

  agg = _edge_agg_kernel(n_pad, e_pad, d_hid)

  hs1 = pl.pallas_call(
      _tc_first,
      out_shape=jax.ShapeDtypeStruct((n, d_hid), jnp.float32),
  )(x, W1, deg_p)

  agg1 = agg(hs1, src_p, dst_p, zeros_d)

  hs2 = pl.pallas_call(
      _tc_mid,
      out_shape=jax.ShapeDtypeStruct((n, d_out), jnp.float32),
  )(agg1, hs1, deg_p, b1r, W2)

  agg2 = agg(hs2, src_p, dst_p, zeros_d)

  out = pl.pallas_call(
      functools.partial(_tc_last, num_graphs=num_graphs),
      out_shape=jax.ShapeDtypeStruct((num_graphs, d_out), jnp.float32),
  )(agg2, hs2, deg_p, b2r, batch2d)

  return out
